# baseline (speedup 1.0000x reference)
import jax
import jax.numpy as jnp
from jax.experimental import pallas as pl
from jax.experimental.pallas import tpu as pltpu

N = 10000
D = 128
H = 128
O = 128

BLOCK = 200
NBLK = N // BLOCK
BLOCK2 = 10000
NBLK2 = N // BLOCK2


def _fused_kernel(adj_ref, feats_ref, wa0t_ref, ba0_ref, wa1t_ref, ba1_ref,
                  wf0t_ref, bf0_ref, wf1t_ref, bf1_ref, attk_ref, vvec_ref,
                  a_ref, h_ref, z_ref, att_ref,
                  a_s, h_s, colsum_s, kvec_s):
    i = pl.program_id(0)

    @pl.when(i < NBLK)
    def _encode():
        a1 = jax.lax.dot_general(adj_ref[...], wa0t_ref[...],
                                 (((1,), (0,)), ((), ())),
                                 preferred_element_type=jnp.float32)
        a1 = jnp.maximum(a1 + ba0_ref[...], 0.0)
        a2 = jax.lax.dot_general(a1, wa1t_ref[...],
                                 (((1,), (0,)), ((), ())),
                                 preferred_element_type=jnp.float32) + ba1_ref[...]

        h1 = jax.lax.dot_general(feats_ref[...], wf0t_ref[...],
                                 (((1,), (0,)), ((), ())),
                                 preferred_element_type=jnp.float32)
        h1 = jnp.maximum(h1 + bf0_ref[...], 0.0)
        h2 = jax.lax.dot_general(h1, wf1t_ref[...],
                                 (((1,), (0,)), ((), ())),
                                 preferred_element_type=jnp.float32) + bf1_ref[...]

        a_ref[...] = a2
        h_ref[...] = h2
        a_s[pl.ds(i * BLOCK, BLOCK), :] = a2
        h_s[pl.ds(i * BLOCK, BLOCK), :] = h2

        part = jnp.sum(a2 + h2, axis=0, keepdims=True)

        @pl.when(i == 0)
        def _():
            colsum_s[...] = part

        @pl.when(i > 0)
        def _():
            colsum_s[...] = colsum_s[...] + part

    @pl.when(i >= NBLK)
    def _finalize():
        @pl.when(i == NBLK)
        def _():
            kvec_s[...] = jax.lax.dot_general(
                colsum_s[...], attk_ref[...],
                (((1,), (0,)), ((), ())),
                preferred_element_type=jnp.float32) / N

        j = i - NBLK
        k_row = kvec_s[...]
        a = a_s[pl.ds(j * BLOCK2, BLOCK2), :]
        h = h_s[pl.ds(j * BLOCK2, BLOCK2), :]
        la = jax.lax.dot_general(k_row, a, (((1,), (1,)), ((), ())),
                                 preferred_element_type=jnp.float32)
        lh = jax.lax.dot_general(k_row, h, (((1,), (1,)), ((), ())),
                                 preferred_element_type=jnp.float32)
        sa = jax.nn.sigmoid(la)
        sh = jax.nn.sigmoid(lh)

        v00 = vvec_ref[0, 0]
        v01 = vvec_ref[0, 1]
        v10 = vvec_ref[0, 2]
        v11 = vvec_ref[0, 3]
        tao = 2.0
        dt = (sa * (v00 - v01) + sh * (v10 - v11)) / tao
        att0_row = jax.nn.sigmoid(dt)
        att0 = att0_row.reshape(BLOCK2, 1)
        att1 = 1.0 - att0

        z_ref[...] = h + att0 * (a - h)
        att_ref[...] = jnp.concatenate([att0, att1], axis=1)


def kernel(adj, feats, Wf0, bf0, Wf1, bf1, Wa0, ba0, Wa1, ba1,
           att_vec_k, att_vec_v):
    wa0t = Wa0.T
    wa1t = Wa1.T
    wf0t = Wf0.T
    wf1t = Wf1.T
    ba0r = ba0.reshape(1, H)
    ba1r = ba1.reshape(1, O)
    bf0r = bf0.reshape(1, H)
    bf1r = bf1.reshape(1, O)
    vvec = att_vec_v.reshape(1, 4)

    full = lambda shape: pl.BlockSpec(shape, lambda i: (0, 0))
    rows1 = lambda shape: pl.BlockSpec(
        shape, lambda i: (jnp.minimum(i, NBLK - 1), 0))
    rows2 = lambda shape: pl.BlockSpec(
        shape, lambda i: (jnp.maximum(i - NBLK, 0), 0))

    a, h, z, att = pl.pallas_call(
        _fused_kernel,
        grid=(NBLK + NBLK2,),
        in_specs=[
            rows1((BLOCK, N)),
            rows1((BLOCK, D)),
            full((N, H)),
            full((1, H)),
            full((H, O)),
            full((1, O)),
            full((D, H)),
            full((1, H)),
            full((H, O)),
            full((1, O)),
            full((O, O)),
            full((1, 4)),
        ],
        out_specs=[
            rows1((BLOCK, O)),
            rows1((BLOCK, O)),
            rows2((BLOCK2, O)),
            rows2((BLOCK2, 2)),
        ],
        out_shape=[
            jax.ShapeDtypeStruct((N, O), jnp.float32),
            jax.ShapeDtypeStruct((N, O), jnp.float32),
            jax.ShapeDtypeStruct((N, O), jnp.float32),
            jax.ShapeDtypeStruct((N, 2), jnp.float32),
        ],
        scratch_shapes=[
            pltpu.VMEM((N, O), jnp.float32),
            pltpu.VMEM((N, O), jnp.float32),
            pltpu.VMEM((1, O), jnp.float32),
            pltpu.VMEM((1, O), jnp.float32),
        ],
        compiler_params=pltpu.CompilerParams(
            dimension_semantics=("arbitrary",),
        ),
    )(adj, feats, wa0t, ba0r, wa1t, ba1r, wf0t, bf0r, wf1t, bf1r,
      att_vec_k, vvec)

    return (a, h, z, att)


# speedup vs baseline: 1.0544x; 1.0544x over previous
import jax
import jax.numpy as jnp
from jax.experimental import pallas as pl
from jax.experimental.pallas import tpu as pltpu

N = 10000
D = 128
H = 128
O = 128

BLOCK = 400
NBLK = N // BLOCK
BLOCK2 = 5000
NBLK2 = N // BLOCK2


def _fused_kernel(adj_ref, feats_ref, wa0t_ref, ba0_ref, wa1t_ref, ba1_ref,
                  wf0t_ref, bf0_ref, wf1t_ref, bf1_ref, attk_ref, vvec_ref,
                  a_ref, h_ref, z_ref, att_ref,
                  a_s, h_s, colsum_s, kvec_s):
    i = pl.program_id(0)

    @pl.when(i < NBLK)
    def _encode():
        a1 = jax.lax.dot_general(adj_ref[...], wa0t_ref[...],
                                 (((1,), (0,)), ((), ())),
                                 preferred_element_type=jnp.float32)
        a1 = jnp.maximum(a1 + ba0_ref[...], 0.0)
        a2 = jax.lax.dot_general(a1, wa1t_ref[...],
                                 (((1,), (0,)), ((), ())),
                                 preferred_element_type=jnp.float32) + ba1_ref[...]

        h1 = jax.lax.dot_general(feats_ref[...], wf0t_ref[...],
                                 (((1,), (0,)), ((), ())),
                                 preferred_element_type=jnp.float32)
        h1 = jnp.maximum(h1 + bf0_ref[...], 0.0)
        h2 = jax.lax.dot_general(h1, wf1t_ref[...],
                                 (((1,), (0,)), ((), ())),
                                 preferred_element_type=jnp.float32) + bf1_ref[...]

        a_ref[...] = a2
        h_ref[...] = h2
        a_s[pl.ds(i * BLOCK, BLOCK), :] = a2
        h_s[pl.ds(i * BLOCK, BLOCK), :] = h2

        part = jnp.sum(a2 + h2, axis=0, keepdims=True)

        @pl.when(i == 0)
        def _():
            colsum_s[...] = part

        @pl.when(i > 0)
        def _():
            colsum_s[...] = colsum_s[...] + part

    @pl.when(i >= NBLK)
    def _finalize():
        @pl.when(i == NBLK)
        def _():
            kvec_s[...] = jax.lax.dot_general(
                colsum_s[...], attk_ref[...],
                (((1,), (0,)), ((), ())),
                preferred_element_type=jnp.float32) / N

        j = i - NBLK
        k_row = kvec_s[...]
        a = a_s[pl.ds(j * BLOCK2, BLOCK2), :]
        h = h_s[pl.ds(j * BLOCK2, BLOCK2), :]
        la = jax.lax.dot_general(k_row, a, (((1,), (1,)), ((), ())),
                                 preferred_element_type=jnp.float32)
        lh = jax.lax.dot_general(k_row, h, (((1,), (1,)), ((), ())),
                                 preferred_element_type=jnp.float32)
        sa = jax.nn.sigmoid(la)
        sh = jax.nn.sigmoid(lh)

        v00 = vvec_ref[0, 0]
        v01 = vvec_ref[0, 1]
        v10 = vvec_ref[0, 2]
        v11 = vvec_ref[0, 3]
        tao = 2.0
        dt = (sa * (v00 - v01) + sh * (v10 - v11)) / tao
        att0_row = jax.nn.sigmoid(dt)
        att0 = att0_row.reshape(BLOCK2, 1)
        att1 = 1.0 - att0

        z_ref[...] = h + att0 * (a - h)
        att_ref[...] = jnp.concatenate([att0, att1], axis=1)


def kernel(adj, feats, Wf0, bf0, Wf1, bf1, Wa0, ba0, Wa1, ba1,
           att_vec_k, att_vec_v):
    wa0t = Wa0.T
    wa1t = Wa1.T
    wf0t = Wf0.T
    wf1t = Wf1.T
    ba0r = ba0.reshape(1, H)
    ba1r = ba1.reshape(1, O)
    bf0r = bf0.reshape(1, H)
    bf1r = bf1.reshape(1, O)
    vvec = att_vec_v.reshape(1, 4)

    full = lambda shape: pl.BlockSpec(shape, lambda i: (0, 0))
    rows1 = lambda shape: pl.BlockSpec(
        shape, lambda i: (jnp.minimum(i, NBLK - 1), 0))
    rows2 = lambda shape: pl.BlockSpec(
        shape, lambda i: (jnp.maximum(i - NBLK, 0), 0))

    a, h, z, att = pl.pallas_call(
        _fused_kernel,
        grid=(NBLK + NBLK2,),
        in_specs=[
            rows1((BLOCK, N)),
            rows1((BLOCK, D)),
            full((N, H)),
            full((1, H)),
            full((H, O)),
            full((1, O)),
            full((D, H)),
            full((1, H)),
            full((H, O)),
            full((1, O)),
            full((O, O)),
            full((1, 4)),
        ],
        out_specs=[
            rows1((BLOCK, O)),
            rows1((BLOCK, O)),
            rows2((BLOCK2, O)),
            rows2((BLOCK2, 2)),
        ],
        out_shape=[
            jax.ShapeDtypeStruct((N, O), jnp.float32),
            jax.ShapeDtypeStruct((N, O), jnp.float32),
            jax.ShapeDtypeStruct((N, O), jnp.float32),
            jax.ShapeDtypeStruct((N, 2), jnp.float32),
        ],
        scratch_shapes=[
            pltpu.VMEM((N, O), jnp.float32),
            pltpu.VMEM((N, O), jnp.float32),
            pltpu.VMEM((1, O), jnp.float32),
            pltpu.VMEM((1, O), jnp.float32),
        ],
        compiler_params=pltpu.CompilerParams(
            dimension_semantics=("arbitrary",),
        ),
    )(adj, feats, wa0t, ba0r, wa1t, ba1r, wf0t, bf0r, wf1t, bf1r,
      att_vec_k, vvec)

    return (a, h, z, att)
